# baseline (speedup 1.0000x reference)
import functools

import jax
import jax.numpy as jnp
from jax import lax
from jax.experimental import pallas as pl
from jax.experimental.pallas import tpu as pltpu
from jax.experimental.pallas import tpu_sc as plsc

N = 10000
E = 320000
H = 128
C = 2

NC = 2
NS = 16
NW = NC * NS
CHUNK = 128
CPW = 80
NCHUNKS_P = NW * CPW
E_PAD = NCHUNKS_P * CHUNK
NBUF = 2
STAGE = 40
NSTAGE = CPW // STAGE
NACC = N + 8
STRIPE = 624
TAIL_BASE = NS * STRIPE
TAIL = N - TAIL_BASE
ZTAIL = NACC - TAIL_BASE

_sc_mesh = plsc.VectorSubcoreMesh(core_axis_name="c", subcore_axis_name="s")


@functools.partial(
    pl.kernel,
    out_type=jax.ShapeDtypeStruct((NC, N, H), jnp.float32),
    mesh=_sc_mesh,
    scratch_types=[
        pltpu.VMEM((STAGE, CHUNK), jnp.int32),
        pltpu.VMEM((STAGE, CHUNK), jnp.int32),
        pltpu.VMEM((NBUF, CHUNK, H), jnp.float32),
        pltpu.VMEM_SHARED((NACC, H), jnp.float32),
    ]
    + [pltpu.SemaphoreType.DMA] * (2 * NBUF),
)
def _sc_aggregate(h_hbm, src_hbm, dst_hbm, zeros_hbm, out_hbm,
                  src_v, dst_v, bufs, agg_sh, *sems):
    gsem = sems[:NBUF]
    ssem = sems[NBUF:]
    c = lax.axis_index("c")
    s = lax.axis_index("s")
    wid = s * NC + c

    base = s * STRIPE
    pltpu.sync_copy(zeros_hbm.at[pl.ds(base, STRIPE)],
                    agg_sh.at[pl.ds(base, STRIPE)])

    @pl.when(s == NS - 1)
    def _():
        pltpu.sync_copy(zeros_hbm.at[pl.ds(TAIL_BASE, ZTAIL)],
                        agg_sh.at[pl.ds(TAIL_BASE, ZTAIL)])

    cbase = wid * CPW
    plsc.subcore_barrier()

    for stage in range(NSTAGE):
        pltpu.sync_copy(src_hbm.at[pl.ds(cbase + stage * STAGE, STAGE)], src_v)
        pltpu.sync_copy(dst_hbm.at[pl.ds(cbase + stage * STAGE, STAGE)], dst_v)

        for b in range(NBUF):
            pltpu.async_copy(h_hbm.at[src_v.at[b]], bufs.at[b], gsem[b])

        for t in range(STAGE):
            b = t % NBUF
            pltpu.make_async_copy(h_hbm.at[src_v.at[t]], bufs.at[b],
                                  gsem[b]).wait()
            pltpu.async_copy(bufs.at[b], agg_sh.at[dst_v.at[t]], ssem[b],
                             add=True)
            pltpu.make_async_copy(bufs.at[b], agg_sh.at[dst_v.at[t]],
                                  ssem[b]).wait()
            if t + NBUF < STAGE:
                pltpu.async_copy(h_hbm.at[src_v.at[t + NBUF]], bufs.at[b],
                                 gsem[b])

    plsc.subcore_barrier()

    pltpu.sync_copy(agg_sh.at[pl.ds(base, STRIPE)],
                    out_hbm.at[c, pl.ds(base, STRIPE)])

    @pl.when(s == NS - 1)
    def _():
        pltpu.sync_copy(agg_sh.at[pl.ds(TAIL_BASE, TAIL)],
                        out_hbm.at[c, pl.ds(TAIL_BASE, TAIL)])


def _tc_layer_body(h_ref, p_ref, wa_ref, ba_ref, wb_ref, bb_ref, out_ref):
    z = h_ref[...] + p_ref[0] + p_ref[1]
    z = jnp.dot(z, wa_ref[...], preferred_element_type=jnp.float32) + ba_ref[...]
    z = jnp.maximum(z, 0.0)
    y = jnp.dot(z, wb_ref[...], preferred_element_type=jnp.float32) + bb_ref[...]
    out_ref[...] = jnp.maximum(y, 0.0)


_tc_layer = pl.pallas_call(
    _tc_layer_body,
    out_shape=jax.ShapeDtypeStruct((N, H), jnp.float32),
)


def _tc_attn_body(x_ref, wta_ref, bta_ref, wtb_ref, btb_ref, wtc_ref, btc_ref,
                  wr_ref, br_ref, wc_ref, bc_ref, out_ref):
    x = x_ref[...]
    a = jnp.tanh(jnp.dot(x, wta_ref[...], preferred_element_type=jnp.float32)
                 + bta_ref[...])
    g = jax.nn.sigmoid(jnp.dot(x, wtb_ref[...], preferred_element_type=jnp.float32)
                       + btb_ref[...])
    s = jnp.dot(a * g, wtc_ref[...], preferred_element_type=jnp.float32) + btc_ref[...]
    s = s[:, :1]
    m = jnp.max(s)
    e = jnp.exp(s - m)
    l = jnp.sum(e)
    hp = jnp.sum(e * x, axis=0, keepdims=True) / l
    h = jnp.maximum(jnp.dot(hp, wr_ref[...], preferred_element_type=jnp.float32)
                    + br_ref[...], 0.0)
    lg = jnp.dot(h, wc_ref[...], preferred_element_type=jnp.float32) + bc_ref[...]
    out_ref[...] = lg


_tc_attn = pl.pallas_call(
    _tc_attn_body,
    out_shape=jax.ShapeDtypeStruct((1, H), jnp.float32),
)


def kernel(x, edge_index, batch, W1a, b1a, W1b, b1b, W2a, b2a, W2b, b2b,
           W3a, b3a, W3b, b3b, Wta, bta, Wtb, btb, Wtc, btc, Wr, br, Wc, bc):
    pad = E_PAD - E
    src = jnp.concatenate(
        [edge_index[0], jnp.zeros((pad,), jnp.int32)]).reshape(NCHUNKS_P, CHUNK)
    dst = jnp.concatenate(
        [edge_index[1], jnp.full((pad,), N, jnp.int32)]).reshape(NCHUNKS_P, CHUNK)
    zeros = jnp.zeros((NACC, H), jnp.float32)

    def gin(h, Wa, ba, Wb, bb):
        p = _sc_aggregate(h, src, dst, zeros)
        return _tc_layer(h, p, Wa, ba.reshape(1, H), Wb, bb.reshape(1, H))

    x1 = gin(x, W1a, b1a, W1b, b1b)
    x2 = gin(x1, W2a, b2a, W2b, b2b)
    x3 = gin(x2, W3a, b3a, W3b, b3b)

    wtc_p = jnp.zeros((H, H), jnp.float32).at[:, :1].set(Wtc)
    btc_p = jnp.zeros((1, H), jnp.float32).at[0, :1].set(btc)
    wc_p = jnp.zeros((H, H), jnp.float32).at[:, :C].set(Wc)
    bc_p = jnp.zeros((1, H), jnp.float32).at[0, :C].set(bc)

    out = _tc_attn(x3, Wta, bta.reshape(1, H), Wtb, btb.reshape(1, H),
                   wtc_p, btc_p, Wr, br.reshape(1, H), wc_p, bc_p)
    return out[:, :C]


# speedup vs baseline: 1.0005x; 1.0005x over previous
import functools

import jax
import jax.numpy as jnp
from jax import lax
from jax.experimental import pallas as pl
from jax.experimental.pallas import tpu as pltpu
from jax.experimental.pallas import tpu_sc as plsc

N = 10000
E = 320000
H = 128
C = 2

NC = 2
NS = 16
NW = NC * NS
CHUNK = 128
CPW = 80
NCHUNKS_P = NW * CPW
E_PAD = NCHUNKS_P * CHUNK
NBUF = 2
STAGE = 40
NSTAGE = CPW // STAGE
NACC = N + 8
STRIPE = 624
TAIL_BASE = NS * STRIPE
TAIL = N - TAIL_BASE
ZTAIL = NACC - TAIL_BASE

_sc_mesh = plsc.VectorSubcoreMesh(core_axis_name="c", subcore_axis_name="s")


@functools.partial(
    pl.kernel,
    out_type=jax.ShapeDtypeStruct((NC, N, H), jnp.float32),
    mesh=_sc_mesh,
    scratch_types=[
        pltpu.VMEM((STAGE, CHUNK), jnp.int32),
        pltpu.VMEM((STAGE, CHUNK), jnp.int32),
        pltpu.VMEM((NBUF, CHUNK, H), jnp.float32),
        pltpu.VMEM_SHARED((NACC, H), jnp.float32),
    ]
    + [pltpu.SemaphoreType.DMA] * (2 * NBUF),
)
def _sc_aggregate(h_hbm, src_hbm, dst_hbm, zeros_hbm, out_hbm,
                  src_v, dst_v, bufs, agg_sh, *sems):
    gsem = sems[:NBUF]
    ssem = sems[NBUF:]
    c = lax.axis_index("c")
    s = lax.axis_index("s")
    wid = c * NS + s

    base = s * STRIPE
    pltpu.sync_copy(zeros_hbm.at[pl.ds(base, STRIPE)],
                    agg_sh.at[pl.ds(base, STRIPE)])

    @pl.when(s == NS - 1)
    def _():
        pltpu.sync_copy(zeros_hbm.at[pl.ds(TAIL_BASE, ZTAIL)],
                        agg_sh.at[pl.ds(TAIL_BASE, ZTAIL)])

    cbase = wid * CPW
    plsc.subcore_barrier()

    for stage in range(NSTAGE):
        pltpu.sync_copy(src_hbm.at[pl.ds(cbase + stage * STAGE, STAGE)], src_v)
        pltpu.sync_copy(dst_hbm.at[pl.ds(cbase + stage * STAGE, STAGE)], dst_v)

        for b in range(NBUF):
            pltpu.async_copy(h_hbm.at[src_v.at[b]], bufs.at[b], gsem[b])

        for t in range(STAGE):
            b = t % NBUF
            pltpu.make_async_copy(h_hbm.at[src_v.at[t]], bufs.at[b],
                                  gsem[b]).wait()
            pltpu.async_copy(bufs.at[b], agg_sh.at[dst_v.at[t]], ssem[b],
                             add=True)
            pltpu.make_async_copy(bufs.at[b], agg_sh.at[dst_v.at[t]],
                                  ssem[b]).wait()
            if t + NBUF < STAGE:
                pltpu.async_copy(h_hbm.at[src_v.at[t + NBUF]], bufs.at[b],
                                 gsem[b])

    plsc.subcore_barrier()

    pltpu.sync_copy(agg_sh.at[pl.ds(base, STRIPE)],
                    out_hbm.at[c, pl.ds(base, STRIPE)])

    @pl.when(s == NS - 1)
    def _():
        pltpu.sync_copy(agg_sh.at[pl.ds(TAIL_BASE, TAIL)],
                        out_hbm.at[c, pl.ds(TAIL_BASE, TAIL)])


def _tc_layer_body(h_ref, p_ref, wa_ref, ba_ref, wb_ref, bb_ref, out_ref):
    z = h_ref[...] + p_ref[0] + p_ref[1]
    z = jnp.dot(z, wa_ref[...], preferred_element_type=jnp.float32) + ba_ref[...]
    z = jnp.maximum(z, 0.0)
    y = jnp.dot(z, wb_ref[...], preferred_element_type=jnp.float32) + bb_ref[...]
    out_ref[...] = jnp.maximum(y, 0.0)


_tc_layer = pl.pallas_call(
    _tc_layer_body,
    out_shape=jax.ShapeDtypeStruct((N, H), jnp.float32),
)


def _tc_attn_body(x_ref, wta_ref, bta_ref, wtb_ref, btb_ref, wtc_ref, btc_ref,
                  wr_ref, br_ref, wc_ref, bc_ref, out_ref):
    x = x_ref[...]
    a = jnp.tanh(jnp.dot(x, wta_ref[...], preferred_element_type=jnp.float32)
                 + bta_ref[...])
    g = jax.nn.sigmoid(jnp.dot(x, wtb_ref[...], preferred_element_type=jnp.float32)
                       + btb_ref[...])
    s = jnp.dot(a * g, wtc_ref[...], preferred_element_type=jnp.float32) + btc_ref[...]
    s = s[:, :1]
    m = jnp.max(s)
    e = jnp.exp(s - m)
    l = jnp.sum(e)
    hp = jnp.sum(e * x, axis=0, keepdims=True) / l
    h = jnp.maximum(jnp.dot(hp, wr_ref[...], preferred_element_type=jnp.float32)
                    + br_ref[...], 0.0)
    lg = jnp.dot(h, wc_ref[...], preferred_element_type=jnp.float32) + bc_ref[...]
    out_ref[...] = lg


_tc_attn = pl.pallas_call(
    _tc_attn_body,
    out_shape=jax.ShapeDtypeStruct((1, H), jnp.float32),
)


def kernel(x, edge_index, batch, W1a, b1a, W1b, b1b, W2a, b2a, W2b, b2b,
           W3a, b3a, W3b, b3b, Wta, bta, Wtb, btb, Wtc, btc, Wr, br, Wc, bc):
    pad = E_PAD - E
    src = jnp.concatenate(
        [edge_index[0], jnp.zeros((pad,), jnp.int32)]).reshape(NCHUNKS_P, CHUNK)
    dst = jnp.concatenate(
        [edge_index[1], jnp.full((pad,), N, jnp.int32)]).reshape(NCHUNKS_P, CHUNK)
    zeros = jnp.zeros((NACC, H), jnp.float32)

    def gin(h, Wa, ba, Wb, bb):
        p = _sc_aggregate(h, src, dst, zeros)
        return _tc_layer(h, p, Wa, ba.reshape(1, H), Wb, bb.reshape(1, H))

    x1 = gin(x, W1a, b1a, W1b, b1b)
    x2 = gin(x1, W2a, b2a, W2b, b2b)
    x3 = gin(x2, W3a, b3a, W3b, b3b)

    wtc_p = jnp.zeros((H, H), jnp.float32).at[:, :1].set(Wtc)
    btc_p = jnp.zeros((1, H), jnp.float32).at[0, :1].set(btc)
    wc_p = jnp.zeros((H, H), jnp.float32).at[:, :C].set(Wc)
    bc_p = jnp.zeros((1, H), jnp.float32).at[0, :C].set(bc)

    out = _tc_attn(x3, Wta, bta.reshape(1, H), Wtb, btb.reshape(1, H),
                   wtc_p, btc_p, Wr, br.reshape(1, H), wc_p, bc_p)
    return out[:, :C]


# speedup vs baseline: 3.4401x; 3.4385x over previous
import functools

import jax
import jax.numpy as jnp
from jax import lax
from jax.experimental import pallas as pl
from jax.experimental.pallas import tpu as pltpu
from jax.experimental.pallas import tpu_sc as plsc

N = 10000
E = 320000
H = 128
C = 2

NC = 2
NS = 16
NW = NC * NS
CHUNK = 128
CPW = 80
NCHUNKS_P = NW * CPW
E_PAD = NCHUNKS_P * CHUNK
NZROWS = 64
NH = N + NZROWS
NBUF = 2
STAGE = 40
NSTAGE = CPW // STAGE
NACC = N + 8
STRIPE = 624
TAIL_BASE = NS * STRIPE
TAIL = N - TAIL_BASE
ZTAIL = NACC - TAIL_BASE

_sc_mesh = plsc.VectorSubcoreMesh(core_axis_name="c", subcore_axis_name="s")


@functools.partial(
    pl.kernel,
    out_type=jax.ShapeDtypeStruct((NC, N, H), jnp.float32),
    mesh=_sc_mesh,
    scratch_types=[
        pltpu.VMEM((STAGE, CHUNK), jnp.int32),
        pltpu.VMEM((STAGE, CHUNK), jnp.int32),
        pltpu.VMEM((NBUF, CHUNK, H), jnp.float32),
        pltpu.VMEM_SHARED((NACC, H), jnp.float32),
    ]
    + [pltpu.SemaphoreType.DMA] * (2 * NBUF),
)
def _sc_aggregate(h_hbm, src_hbm, dst_hbm, zeros_hbm, out_hbm,
                  src_v, dst_v, bufs, agg_sh, *sems):
    gsem = sems[:NBUF]
    ssem = sems[NBUF:]
    c = lax.axis_index("c")
    s = lax.axis_index("s")
    wid = c * NS + s

    base = s * STRIPE
    pltpu.sync_copy(zeros_hbm.at[pl.ds(base, STRIPE)],
                    agg_sh.at[pl.ds(base, STRIPE)])

    @pl.when(s == NS - 1)
    def _():
        pltpu.sync_copy(zeros_hbm.at[pl.ds(TAIL_BASE, ZTAIL)],
                        agg_sh.at[pl.ds(TAIL_BASE, ZTAIL)])

    cbase = wid * CPW
    plsc.subcore_barrier()

    for stage in range(NSTAGE):
        pltpu.sync_copy(src_hbm.at[pl.ds(cbase + stage * STAGE, STAGE)], src_v)
        pltpu.sync_copy(dst_hbm.at[pl.ds(cbase + stage * STAGE, STAGE)], dst_v)

        for b in range(NBUF):
            pltpu.async_copy(h_hbm.at[src_v.at[b]], bufs.at[b], gsem[b])

        for t in range(STAGE):
            b = t % NBUF
            pltpu.make_async_copy(h_hbm.at[src_v.at[t]], bufs.at[b],
                                  gsem[b]).wait()
            pltpu.async_copy(bufs.at[b], agg_sh.at[dst_v.at[t]], ssem[b],
                             add=True)
            pltpu.make_async_copy(bufs.at[b], agg_sh.at[dst_v.at[t]],
                                  ssem[b]).wait()
            if t + NBUF < STAGE:
                pltpu.async_copy(h_hbm.at[src_v.at[t + NBUF]], bufs.at[b],
                                 gsem[b])

    plsc.subcore_barrier()

    pltpu.sync_copy(agg_sh.at[pl.ds(base, STRIPE)],
                    out_hbm.at[c, pl.ds(base, STRIPE)])

    @pl.when(s == NS - 1)
    def _():
        pltpu.sync_copy(agg_sh.at[pl.ds(TAIL_BASE, TAIL)],
                        out_hbm.at[c, pl.ds(TAIL_BASE, TAIL)])


def _tc_layer_body(h_ref, p_ref, wa_ref, ba_ref, wb_ref, bb_ref, out_ref):
    z = h_ref[0:N, :] + p_ref[0] + p_ref[1]
    z = jnp.dot(z, wa_ref[...], preferred_element_type=jnp.float32) + ba_ref[...]
    z = jnp.maximum(z, 0.0)
    y = jnp.dot(z, wb_ref[...], preferred_element_type=jnp.float32) + bb_ref[...]
    out_ref[...] = jnp.maximum(y, 0.0)


_tc_layer = pl.pallas_call(
    _tc_layer_body,
    out_shape=jax.ShapeDtypeStruct((N, H), jnp.float32),
)


def _tc_attn_body(x_ref, wta_ref, bta_ref, wtb_ref, btb_ref, wtc_ref, btc_ref,
                  wr_ref, br_ref, wc_ref, bc_ref, out_ref):
    x = x_ref[...]
    a = jnp.tanh(jnp.dot(x, wta_ref[...], preferred_element_type=jnp.float32)
                 + bta_ref[...])
    g = jax.nn.sigmoid(jnp.dot(x, wtb_ref[...], preferred_element_type=jnp.float32)
                       + btb_ref[...])
    s = jnp.dot(a * g, wtc_ref[...], preferred_element_type=jnp.float32) + btc_ref[...]
    s = s[:, :1]
    m = jnp.max(s)
    e = jnp.exp(s - m)
    l = jnp.sum(e)
    hp = jnp.sum(e * x, axis=0, keepdims=True) / l
    h = jnp.maximum(jnp.dot(hp, wr_ref[...], preferred_element_type=jnp.float32)
                    + br_ref[...], 0.0)
    lg = jnp.dot(h, wc_ref[...], preferred_element_type=jnp.float32) + bc_ref[...]
    out_ref[...] = lg


_tc_attn = pl.pallas_call(
    _tc_attn_body,
    out_shape=jax.ShapeDtypeStruct((1, H), jnp.float32),
)


def kernel(x, edge_index, batch, W1a, b1a, W1b, b1b, W2a, b2a, W2b, b2b,
           W3a, b3a, W3b, b3b, Wta, bta, Wtb, btb, Wtc, btc, Wr, br, Wc, bc):
    pad = E_PAD - E
    pidx = jnp.arange(pad, dtype=jnp.int32)
    src = jnp.concatenate(
        [edge_index[0], N + pidx % NZROWS]).reshape(NCHUNKS_P, CHUNK)
    dst = jnp.concatenate(
        [edge_index[1], pidx % N]).reshape(NCHUNKS_P, CHUNK)
    zeros = jnp.zeros((NACC, H), jnp.float32)
    zrows = jnp.zeros((NZROWS, H), jnp.float32)

    def gin(h, Wa, ba, Wb, bb):
        hp = jnp.concatenate([h, zrows])
        p = _sc_aggregate(hp, src, dst, zeros)
        return _tc_layer(hp, p, Wa, ba.reshape(1, H), Wb, bb.reshape(1, H))

    x1 = gin(x, W1a, b1a, W1b, b1b)
    x2 = gin(x1, W2a, b2a, W2b, b2b)
    x3 = gin(x2, W3a, b3a, W3b, b3b)

    wtc_p = jnp.zeros((H, H), jnp.float32).at[:, :1].set(Wtc)
    btc_p = jnp.zeros((1, H), jnp.float32).at[0, :1].set(btc)
    wc_p = jnp.zeros((H, H), jnp.float32).at[:, :C].set(Wc)
    bc_p = jnp.zeros((1, H), jnp.float32).at[0, :C].set(bc)

    out = _tc_attn(x3, Wta, bta.reshape(1, H), Wtb, btb.reshape(1, H),
                   wtc_p, btc_p, Wr, br.reshape(1, H), wc_p, bc_p)
    return out[:, :C]


# speedup vs baseline: 3.5262x; 1.0251x over previous
import functools

import jax
import jax.numpy as jnp
from jax import lax
from jax.experimental import pallas as pl
from jax.experimental.pallas import tpu as pltpu
from jax.experimental.pallas import tpu_sc as plsc

N = 10000
E = 320000
H = 128
C = 2

NC = 2
NS = 16
NW = NC * NS
CHUNK = 128
CPW = 80
NCHUNKS_P = NW * CPW
E_PAD = NCHUNKS_P * CHUNK
NZROWS = 64
NH = N + NZROWS
NBUF = 2
STAGE = 40
NSTAGE = CPW // STAGE
NACC = N + 8
STRIPE = 624
TAIL_BASE = NS * STRIPE
TAIL = N - TAIL_BASE
ZTAIL = NACC - TAIL_BASE

_sc_mesh = plsc.VectorSubcoreMesh(core_axis_name="c", subcore_axis_name="s")


@functools.partial(
    pl.kernel,
    out_type=jax.ShapeDtypeStruct((NC, N, H), jnp.float32),
    mesh=_sc_mesh,
    scratch_types=[
        pltpu.VMEM((STAGE, CHUNK), jnp.int32),
        pltpu.VMEM((STAGE, CHUNK), jnp.int32),
        pltpu.VMEM((NBUF, CHUNK, H), jnp.float32),
        pltpu.VMEM_SHARED((NACC, H), jnp.float32),
    ]
    + [pltpu.SemaphoreType.DMA] * (2 * NBUF),
)
def _sc_aggregate(h_hbm, src_hbm, dst_hbm, zeros_hbm, out_hbm,
                  src_v, dst_v, bufs, agg_sh, *sems):
    gsem = sems[:NBUF]
    ssem = sems[NBUF:]
    c = lax.axis_index("c")
    s = lax.axis_index("s")
    wid = c * NS + s

    base = s * STRIPE
    pltpu.sync_copy(zeros_hbm.at[pl.ds(base, STRIPE)],
                    agg_sh.at[pl.ds(base, STRIPE)])

    @pl.when(s == NS - 1)
    def _():
        pltpu.sync_copy(zeros_hbm.at[pl.ds(TAIL_BASE, ZTAIL)],
                        agg_sh.at[pl.ds(TAIL_BASE, ZTAIL)])

    cbase = wid * CPW
    plsc.subcore_barrier()

    for stage in range(NSTAGE):
        pltpu.sync_copy(src_hbm.at[pl.ds(cbase + stage * STAGE, STAGE)], src_v)
        pltpu.sync_copy(dst_hbm.at[pl.ds(cbase + stage * STAGE, STAGE)], dst_v)

        for b in range(NBUF):
            pltpu.async_copy(h_hbm.at[src_v.at[b]], bufs.at[b], gsem[b])

        for t in range(STAGE):
            b = t % NBUF
            pltpu.make_async_copy(h_hbm.at[src_v.at[t]], bufs.at[b],
                                  gsem[b]).wait()
            pltpu.async_copy(bufs.at[b], agg_sh.at[dst_v.at[t]], ssem[b],
                             add=True)
            pltpu.make_async_copy(bufs.at[b], agg_sh.at[dst_v.at[t]],
                                  ssem[b]).wait()
            if t + NBUF < STAGE:
                pltpu.async_copy(h_hbm.at[src_v.at[t + NBUF]], bufs.at[b],
                                 gsem[b])

    plsc.subcore_barrier()

    pltpu.sync_copy(agg_sh.at[pl.ds(base, STRIPE)],
                    out_hbm.at[c, pl.ds(base, STRIPE)])

    @pl.when(s == NS - 1)
    def _():
        pltpu.sync_copy(agg_sh.at[pl.ds(TAIL_BASE, TAIL)],
                        out_hbm.at[c, pl.ds(TAIL_BASE, TAIL)])


def _tc_layer_mlp(h_ref, p_ref, wa_ref, ba_ref, wb_ref, bb_ref):
    z = h_ref[0:N, :] + p_ref[0] + p_ref[1]
    z = jnp.dot(z, wa_ref[...], preferred_element_type=jnp.float32) + ba_ref[...]
    z = jnp.maximum(z, 0.0)
    y = jnp.dot(z, wb_ref[...], preferred_element_type=jnp.float32) + bb_ref[...]
    return jnp.maximum(y, 0.0)


def _tc_layer_pad_body(h_ref, p_ref, wa_ref, ba_ref, wb_ref, bb_ref, out_ref):
    out_ref[0:N, :] = _tc_layer_mlp(h_ref, p_ref, wa_ref, ba_ref, wb_ref, bb_ref)
    out_ref[N:NH, :] = jnp.zeros((NZROWS, H), jnp.float32)


_tc_layer_pad = pl.pallas_call(
    _tc_layer_pad_body,
    out_shape=jax.ShapeDtypeStruct((NH, H), jnp.float32),
)


def _tc_layer_body(h_ref, p_ref, wa_ref, ba_ref, wb_ref, bb_ref, out_ref):
    out_ref[...] = _tc_layer_mlp(h_ref, p_ref, wa_ref, ba_ref, wb_ref, bb_ref)


_tc_layer = pl.pallas_call(
    _tc_layer_body,
    out_shape=jax.ShapeDtypeStruct((N, H), jnp.float32),
)


def _tc_attn_body(x_ref, wta_ref, bta_ref, wtb_ref, btb_ref, wtc_ref, btc_ref,
                  wr_ref, br_ref, wc_ref, bc_ref, out_ref):
    x = x_ref[...]
    a = jnp.tanh(jnp.dot(x, wta_ref[...], preferred_element_type=jnp.float32)
                 + bta_ref[...])
    g = jax.nn.sigmoid(jnp.dot(x, wtb_ref[...], preferred_element_type=jnp.float32)
                       + btb_ref[...])
    s = jnp.dot(a * g, wtc_ref[...], preferred_element_type=jnp.float32) + btc_ref[...]
    s = s[:, :1]
    m = jnp.max(s)
    e = jnp.exp(s - m)
    l = jnp.sum(e)
    hp = jnp.sum(e * x, axis=0, keepdims=True) / l
    h = jnp.maximum(jnp.dot(hp, wr_ref[...], preferred_element_type=jnp.float32)
                    + br_ref[...], 0.0)
    lg = jnp.dot(h, wc_ref[...], preferred_element_type=jnp.float32) + bc_ref[...]
    out_ref[...] = lg


_tc_attn = pl.pallas_call(
    _tc_attn_body,
    out_shape=jax.ShapeDtypeStruct((1, H), jnp.float32),
)


def kernel(x, edge_index, batch, W1a, b1a, W1b, b1b, W2a, b2a, W2b, b2b,
           W3a, b3a, W3b, b3b, Wta, bta, Wtb, btb, Wtc, btc, Wr, br, Wc, bc):
    pad = E_PAD - E
    pidx = jnp.arange(pad, dtype=jnp.int32)
    src = jnp.concatenate(
        [edge_index[0], N + pidx % NZROWS]).reshape(NCHUNKS_P, CHUNK)
    dst = jnp.concatenate(
        [edge_index[1], pidx % N]).reshape(NCHUNKS_P, CHUNK)
    zeros = jnp.zeros((NACC, H), jnp.float32)
    zrows = jnp.zeros((NZROWS, H), jnp.float32)

    def gin(hp, layer_call, Wa, ba, Wb, bb):
        p = _sc_aggregate(hp, src, dst, zeros)
        return layer_call(hp, p, Wa, ba.reshape(1, H), Wb, bb.reshape(1, H))

    x0p = jnp.concatenate([x, zrows])
    x1p = gin(x0p, _tc_layer_pad, W1a, b1a, W1b, b1b)
    x2p = gin(x1p, _tc_layer_pad, W2a, b2a, W2b, b2b)
    x3 = gin(x2p, _tc_layer, W3a, b3a, W3b, b3b)

    wtc_p = jnp.zeros((H, H), jnp.float32).at[:, :1].set(Wtc)
    btc_p = jnp.zeros((1, H), jnp.float32).at[0, :1].set(btc)
    wc_p = jnp.zeros((H, H), jnp.float32).at[:, :C].set(Wc)
    bc_p = jnp.zeros((1, H), jnp.float32).at[0, :C].set(bc)

    out = _tc_attn(x3, Wta, bta.reshape(1, H), Wtb, btb.reshape(1, H),
                   wtc_p, btc_p, Wr, br.reshape(1, H), wc_p, bc_p)
    return out[:, :C]
